# baseline (speedup 1.0000x reference)
import jax
import jax.numpy as jnp
from jax.experimental import pallas as pl
from jax.experimental.pallas import tpu as pltpu

_B, _T, _S, _H = 16, 4, 1025, 1280


def _add_body(ids_ref, hs_ref, emb_ref, out_ref):
    del ids_ref
    out_ref[...] = hs_ref[...] + emb_ref[...]


def kernel(hidden_states, aspect_ratio_ids, embedding_weight):
    ids = aspect_ratio_ids.astype(jnp.int32)
    emb = embedding_weight.reshape(-1, _T, 1, _H)

    grid_spec = pltpu.PrefetchScalarGridSpec(
        num_scalar_prefetch=1,
        grid=(_B, _T),
        in_specs=[
            pl.BlockSpec((1, 1, _S, _H), lambda b, t, ids_ref: (b, t, 0, 0)),
            pl.BlockSpec((1, 1, 1, _H), lambda b, t, ids_ref: (ids_ref[b], t, 0, 0)),
        ],
        out_specs=pl.BlockSpec((1, 1, _S, _H), lambda b, t, ids_ref: (b, t, 0, 0)),
    )
    return pl.pallas_call(
        _add_body,
        grid_spec=grid_spec,
        out_shape=jax.ShapeDtypeStruct((_B, _T, _S, _H), jnp.float32),
    )(ids, hidden_states, emb)
